# baseline (speedup 1.0000x reference)
import functools

import jax
import jax.numpy as jnp
from jax import lax
from jax.experimental import pallas as pl
from jax.experimental.pallas import tpu as pltpu
from jax.experimental.pallas import tpu_sc as plsc

N = 10000
NPAD = 10240
E = 320000
D = 128
G = 64
NC = 2
NS = 16
EK = 128
EPAD = 327680
NCHUNK = 80
WE = EPAD // (NC * NS)
NACC = 10008
RPS = 640
RLAST = N - (NS - 1) * RPS
EDGES_PER_SUB = E // (NC * NS)
NB = 10
BN = N // NB


def _sc_mesh():
    return plsc.VectorSubcoreMesh(core_axis_name="c", subcore_axis_name="s")


def _row_slab(s, copy_fn):
    @pl.when(s < NS - 1)
    def _():
        copy_fn(s * RPS, RPS)

    @pl.when(s == NS - 1)
    def _():
        copy_fn((NS - 1) * RPS, RLAST)


def _deg_body(dst_hbm, out_hbm, didx_v, hist_v):
    c = lax.axis_index("c")
    s = lax.axis_index("s")
    zeros16 = jnp.zeros((16,), jnp.float32)

    def zstep(i, carry):
        hist_v[pl.ds(i * 16, 16)] = zeros16
        return carry

    lax.fori_loop(0, NPAD // 16, zstep, 0)
    base = c * (E // NC) + s * EDGES_PER_SUB
    pltpu.sync_copy(dst_hbm.at[pl.ds(base, EDGES_PER_SUB)], didx_v)
    ones16 = jnp.ones((16,), jnp.float32)

    def step(i, carry):
        idx = didx_v[pl.ds(i * 16, 16)]
        plsc.addupdate_scatter(hist_v, [idx], ones16)
        return carry

    lax.fori_loop(0, EDGES_PER_SUB // 16, step, 0)
    pltpu.sync_copy(hist_v, out_hbm.at[c, s])


def _sc_degree(dst):
    return pl.kernel(
        _deg_body,
        out_type=jax.ShapeDtypeStruct((NC, NS, NPAD), jnp.float32),
        mesh=_sc_mesh(),
        compiler_params=pltpu.CompilerParams(needs_layout_passes=False),
        scratch_types=[
            pltpu.VMEM((EDGES_PER_SUB,), jnp.int32),
            pltpu.VMEM((NPAD,), jnp.float32),
        ],
    )(dst)


def _scatter_body(g_hbm, srcp_hbm, dstp_hbm, out_hbm, sidx_v, didx_v,
                  rows_v, sem, acc_sh):
    c = lax.axis_index("c")
    s = lax.axis_index("s")
    w = c * NS + s
    _row_slab(s, lambda r0, nr: pltpu.sync_copy(
        g_hbm.at[pl.ds(r0, nr)], acc_sh.at[pl.ds(r0, nr)]))
    plsc.subcore_barrier()
    base = w * WE

    def step(j, carry):
        pltpu.sync_copy(srcp_hbm.at[pl.ds(base + j * EK, EK)], sidx_v)
        pltpu.sync_copy(dstp_hbm.at[pl.ds(base + j * EK, EK)], didx_v)
        pltpu.async_copy(g_hbm.at[sidx_v], rows_v, sem).wait()
        pltpu.sync_copy(rows_v, acc_sh.at[didx_v], add=True)
        return carry

    lax.fori_loop(0, NCHUNK, step, 0)
    plsc.subcore_barrier()
    _row_slab(s, lambda r0, nr: pltpu.sync_copy(
        acc_sh.at[pl.ds(r0, nr)], out_hbm.at[c, pl.ds(r0, nr)]))


def _sc_scatter(g, srcp, dstp):
    return pl.kernel(
        _scatter_body,
        out_type=jax.ShapeDtypeStruct((NC, N, D), jnp.float32),
        mesh=_sc_mesh(),
        scratch_types=[
            pltpu.VMEM((EK,), jnp.int32),
            pltpu.VMEM((EK,), jnp.int32),
            pltpu.VMEM((EK, D), jnp.float32),
            pltpu.SemaphoreType.DMA,
            pltpu.VMEM_SHARED((NACC, D), jnp.float32),
        ],
    )(g, srcp, dstp)


def _tc1_body(x_ref, w_ref, deg_ref, g_ref):
    deg = jnp.sum(deg_ref[...], axis=0) + 1.0
    dinv = lax.rsqrt(deg)
    h = jnp.dot(x_ref[...], w_ref[...], preferred_element_type=jnp.float32)
    g_ref[...] = h * dinv


def _tc_stage1(x, W1, deg2):
    return pl.pallas_call(
        _tc1_body,
        grid=(NB,),
        in_specs=[
            pl.BlockSpec((BN, D), lambda i: (i, 0)),
            pl.BlockSpec((D, D), lambda i: (0, 0)),
            pl.BlockSpec((NC * NS, BN, 1), lambda i: (0, i, 0)),
        ],
        out_specs=pl.BlockSpec((BN, D), lambda i: (i, 0)),
        out_shape=jax.ShapeDtypeStruct((N, D), jnp.float32),
    )(x, W1, deg2)


def _tc2_body(acc_ref, g1_ref, deg_ref, w2_ref, b1_ref, batch_ref,
              g2_ref, p1_ref, p1_acc):
    i = pl.program_id(0)
    deg = jnp.sum(deg_ref[...], axis=0) + 1.0
    dinv = lax.rsqrt(deg)
    esum = acc_ref[0] + acc_ref[1] - g1_ref[...]
    out1 = jax.nn.relu(esum * dinv + b1_ref[...])
    h2 = jnp.dot(out1, w2_ref[...], preferred_element_type=jnp.float32)
    g2_ref[...] = h2 * dinv
    onehot = (batch_ref[0] == lax.broadcasted_iota(jnp.int32, (G, BN), 0)
              ).astype(jnp.float32)
    part = jnp.dot(onehot, out1, preferred_element_type=jnp.float32)

    @pl.when(i == 0)
    def _():
        p1_acc[...] = jnp.zeros_like(p1_acc)

    p1_acc[...] += part

    @pl.when(i == NB - 1)
    def _():
        p1_ref[...] = p1_acc[...]


def _tc_stage2(acc1, g1, deg2, W2, b1r, batch3):
    return pl.pallas_call(
        _tc2_body,
        grid=(NB,),
        in_specs=[
            pl.BlockSpec((NC, BN, D), lambda i: (0, i, 0)),
            pl.BlockSpec((BN, D), lambda i: (i, 0)),
            pl.BlockSpec((NC * NS, BN, 1), lambda i: (0, i, 0)),
            pl.BlockSpec((D, D), lambda i: (0, 0)),
            pl.BlockSpec((1, D), lambda i: (0, 0)),
            pl.BlockSpec((1, 1, BN), lambda i: (i, 0, 0)),
        ],
        out_specs=[
            pl.BlockSpec((BN, D), lambda i: (i, 0)),
            pl.BlockSpec((G, D), lambda i: (0, 0)),
        ],
        out_shape=[
            jax.ShapeDtypeStruct((N, D), jnp.float32),
            jax.ShapeDtypeStruct((G, D), jnp.float32),
        ],
        scratch_shapes=[pltpu.VMEM((G, D), jnp.float32)],
    )(acc1, g1, deg2, W2, b1r, batch3)


def _tc3_body(acc_ref, g2_ref, deg_ref, b2_ref, batch_ref, p1_ref,
              wl1_ref, bl1_ref, wl2_ref, bl2_ref, h_ref, lsm_ref, p2_acc):
    i = pl.program_id(0)
    deg = jnp.sum(deg_ref[...], axis=0) + 1.0
    dinv = lax.rsqrt(deg)
    esum = acc_ref[0] + acc_ref[1] - g2_ref[...]
    out2 = jax.nn.relu(esum * dinv + b2_ref[...])
    onehot = (batch_ref[0] == lax.broadcasted_iota(jnp.int32, (G, BN), 0)
              ).astype(jnp.float32)
    part = jnp.dot(onehot, out2, preferred_element_type=jnp.float32)

    @pl.when(i == 0)
    def _():
        p2_acc[...] = jnp.zeros_like(p2_acc)

    p2_acc[...] += part

    @pl.when(i == NB - 1)
    def _():
        p = jnp.concatenate([p1_ref[...], p2_acc[...]], axis=1)
        h = jnp.dot(p, wl1_ref[...], preferred_element_type=jnp.float32)
        h = jax.nn.relu(h + bl1_ref[...])
        h = jnp.dot(h, wl2_ref[...], preferred_element_type=jnp.float32)
        h = h + bl2_ref[...]
        m = jnp.max(h, axis=1, keepdims=True)
        lse = jnp.log(jnp.sum(jnp.exp(h - m), axis=1, keepdims=True))
        h_ref[...] = h
        lsm_ref[...] = h - m - lse


def _tc_stage3(acc2, g2, deg2, b2r, batch3, p1, Wl1, bl1r, Wl2, bl2r):
    return pl.pallas_call(
        _tc3_body,
        grid=(NB,),
        in_specs=[
            pl.BlockSpec((NC, BN, D), lambda i: (0, i, 0)),
            pl.BlockSpec((BN, D), lambda i: (i, 0)),
            pl.BlockSpec((NC * NS, BN, 1), lambda i: (0, i, 0)),
            pl.BlockSpec((1, D), lambda i: (0, 0)),
            pl.BlockSpec((1, 1, BN), lambda i: (i, 0, 0)),
            pl.BlockSpec((G, D), lambda i: (0, 0)),
            pl.BlockSpec((2 * D, 2 * D), lambda i: (0, 0)),
            pl.BlockSpec((1, 2 * D), lambda i: (0, 0)),
            pl.BlockSpec((2 * D, 10), lambda i: (0, 0)),
            pl.BlockSpec((1, 10), lambda i: (0, 0)),
        ],
        out_specs=[
            pl.BlockSpec((G, 10), lambda i: (0, 0)),
            pl.BlockSpec((G, 10), lambda i: (0, 0)),
        ],
        out_shape=[
            jax.ShapeDtypeStruct((G, 10), jnp.float32),
            jax.ShapeDtypeStruct((G, 10), jnp.float32),
        ],
        scratch_shapes=[pltpu.VMEM((G, D), jnp.float32)],
    )(acc2, g2, deg2, b2r, batch3, p1, Wl1, bl1r, Wl2, bl2r)


def kernel(x, edge_index, batch, W1, b1, W2, b2, Wl1, bl1, Wl2, bl2):
    src = edge_index[0]
    dst = edge_index[1]
    batch3 = jnp.reshape(batch, (NB, 1, BN))

    srcp = jnp.pad(src, (0, EPAD - E))
    dstp = jnp.pad(dst, (0, EPAD - E), constant_values=N)

    deg2 = jnp.reshape(_sc_degree(dst), (NC * NS, NPAD, 1))
    g1 = _tc_stage1(x, W1, deg2)
    acc1 = _sc_scatter(g1, srcp, dstp)
    g2, p1 = _tc_stage2(acc1, g1, deg2, W2, jnp.reshape(b1, (1, D)), batch3)
    acc2 = _sc_scatter(g2, srcp, dstp)
    h, lsm = _tc_stage3(acc2, g2, deg2, jnp.reshape(b2, (1, D)), batch3, p1,
                        Wl1, jnp.reshape(bl1, (1, 2 * D)), Wl2,
                        jnp.reshape(bl2, (1, 10)))
    return (h, lsm)


# speedup vs baseline: 1.0037x; 1.0037x over previous
import functools

import jax
import jax.numpy as jnp
from jax import lax
from jax.experimental import pallas as pl
from jax.experimental.pallas import tpu as pltpu
from jax.experimental.pallas import tpu_sc as plsc

N = 10000
NPAD = 10240
E = 320000
D = 128
G = 64
NC = 2
NS = 16
EK = 128
EPAD = 327680
NCHUNK = 80
WE = EPAD // (NC * NS)
NACC = 10240
RPS = 640
RLAST = N - (NS - 1) * RPS
EDGES_PER_SUB = E // (NC * NS)
NB = 10
BN = N // NB


def _sc_mesh():
    return plsc.VectorSubcoreMesh(core_axis_name="c", subcore_axis_name="s")


def _row_slab(s, copy_fn):
    @pl.when(s < NS - 1)
    def _():
        copy_fn(s * RPS, RPS)

    @pl.when(s == NS - 1)
    def _():
        copy_fn((NS - 1) * RPS, RLAST)


def _deg_body(dst_hbm, out_hbm, didx_v, hist_v):
    c = lax.axis_index("c")
    s = lax.axis_index("s")
    zeros16 = jnp.zeros((16,), jnp.float32)

    def zstep(i, carry):
        hist_v[pl.ds(i * 16, 16)] = zeros16
        return carry

    lax.fori_loop(0, NPAD // 16, zstep, 0)
    base = c * (E // NC) + s * EDGES_PER_SUB
    pltpu.sync_copy(dst_hbm.at[pl.ds(base, EDGES_PER_SUB)], didx_v)
    ones16 = jnp.ones((16,), jnp.float32)

    def step(i, carry):
        idx = didx_v[pl.ds(i * 16, 16)]
        plsc.addupdate_scatter(hist_v, [idx], ones16)
        return carry

    lax.fori_loop(0, EDGES_PER_SUB // 16, step, 0)
    pltpu.sync_copy(hist_v, out_hbm.at[c, s])


def _sc_degree(dst):
    return pl.kernel(
        _deg_body,
        out_type=jax.ShapeDtypeStruct((NC, NS, NPAD), jnp.float32),
        mesh=_sc_mesh(),
        compiler_params=pltpu.CompilerParams(needs_layout_passes=False),
        scratch_types=[
            pltpu.VMEM((EDGES_PER_SUB,), jnp.int32),
            pltpu.VMEM((NPAD,), jnp.float32),
        ],
    )(dst)


def _scatter_body(g_hbm, srcp_hbm, dstp_hbm, out_hbm, sidx_v, didx_v,
                  rows_v, sem, acc_sh):
    c = lax.axis_index("c")
    s = lax.axis_index("s")
    w = c * NS + s
    _row_slab(s, lambda r0, nr: pltpu.sync_copy(
        g_hbm.at[pl.ds(r0, nr)], acc_sh.at[pl.ds(r0, nr)]))
    plsc.subcore_barrier()
    base = w * WE

    def step(j, carry):
        pltpu.sync_copy(srcp_hbm.at[pl.ds(base + j * EK, EK)], sidx_v)
        pltpu.sync_copy(dstp_hbm.at[pl.ds(base + j * EK, EK)], didx_v)
        pltpu.async_copy(g_hbm.at[sidx_v], rows_v, sem).wait()
        pltpu.sync_copy(rows_v, acc_sh.at[didx_v], add=True)
        return carry

    lax.fori_loop(0, NCHUNK, step, 0)
    plsc.subcore_barrier()
    _row_slab(s, lambda r0, nr: pltpu.sync_copy(
        acc_sh.at[pl.ds(r0, nr)], out_hbm.at[c, pl.ds(r0, nr)]))


def _sc_scatter(g, srcp, dstp):
    return pl.kernel(
        _scatter_body,
        out_type=jax.ShapeDtypeStruct((NC, N, D), jnp.float32),
        mesh=_sc_mesh(),
        scratch_types=[
            pltpu.VMEM((EK,), jnp.int32),
            pltpu.VMEM((EK,), jnp.int32),
            pltpu.VMEM((EK, D), jnp.float32),
            pltpu.SemaphoreType.DMA,
            pltpu.VMEM_SHARED((NACC, D), jnp.float32),
        ],
    )(g, srcp, dstp)


def _tc1_body(x_ref, w_ref, deg_ref, g_ref):
    deg = jnp.sum(deg_ref[...], axis=0) + 1.0
    dinv = lax.rsqrt(deg)
    h = jnp.dot(x_ref[...], w_ref[...], preferred_element_type=jnp.float32)
    g_ref[...] = h * dinv


def _tc_stage1(x, W1, deg2):
    return pl.pallas_call(
        _tc1_body,
        grid=(NB,),
        in_specs=[
            pl.BlockSpec((BN, D), lambda i: (i, 0)),
            pl.BlockSpec((D, D), lambda i: (0, 0)),
            pl.BlockSpec((NC * NS, BN, 1), lambda i: (0, i, 0)),
        ],
        out_specs=pl.BlockSpec((BN, D), lambda i: (i, 0)),
        out_shape=jax.ShapeDtypeStruct((N, D), jnp.float32),
    )(x, W1, deg2)


def _tc2_body(acc_ref, g1_ref, deg_ref, w2_ref, b1_ref, batch_ref,
              g2_ref, p1_ref, p1_acc):
    i = pl.program_id(0)
    deg = jnp.sum(deg_ref[...], axis=0) + 1.0
    dinv = lax.rsqrt(deg)
    esum = acc_ref[0] + acc_ref[1] - g1_ref[...]
    out1 = jax.nn.relu(esum * dinv + b1_ref[...])
    h2 = jnp.dot(out1, w2_ref[...], preferred_element_type=jnp.float32)
    g2_ref[...] = h2 * dinv
    onehot = (batch_ref[0] == lax.broadcasted_iota(jnp.int32, (G, BN), 0)
              ).astype(jnp.float32)
    part = jnp.dot(onehot, out1, preferred_element_type=jnp.float32)

    @pl.when(i == 0)
    def _():
        p1_acc[...] = jnp.zeros_like(p1_acc)

    p1_acc[...] += part

    @pl.when(i == NB - 1)
    def _():
        p1_ref[...] = p1_acc[...]


def _tc_stage2(acc1, g1, deg2, W2, b1r, batch3):
    return pl.pallas_call(
        _tc2_body,
        grid=(NB,),
        in_specs=[
            pl.BlockSpec((NC, BN, D), lambda i: (0, i, 0)),
            pl.BlockSpec((BN, D), lambda i: (i, 0)),
            pl.BlockSpec((NC * NS, BN, 1), lambda i: (0, i, 0)),
            pl.BlockSpec((D, D), lambda i: (0, 0)),
            pl.BlockSpec((1, D), lambda i: (0, 0)),
            pl.BlockSpec((1, 1, BN), lambda i: (i, 0, 0)),
        ],
        out_specs=[
            pl.BlockSpec((BN, D), lambda i: (i, 0)),
            pl.BlockSpec((G, D), lambda i: (0, 0)),
        ],
        out_shape=[
            jax.ShapeDtypeStruct((N, D), jnp.float32),
            jax.ShapeDtypeStruct((G, D), jnp.float32),
        ],
        scratch_shapes=[pltpu.VMEM((G, D), jnp.float32)],
    )(acc1, g1, deg2, W2, b1r, batch3)


def _tc3_body(acc_ref, g2_ref, deg_ref, b2_ref, batch_ref, p1_ref,
              wl1_ref, bl1_ref, wl2_ref, bl2_ref, h_ref, lsm_ref, p2_acc):
    i = pl.program_id(0)
    deg = jnp.sum(deg_ref[...], axis=0) + 1.0
    dinv = lax.rsqrt(deg)
    esum = acc_ref[0] + acc_ref[1] - g2_ref[...]
    out2 = jax.nn.relu(esum * dinv + b2_ref[...])
    onehot = (batch_ref[0] == lax.broadcasted_iota(jnp.int32, (G, BN), 0)
              ).astype(jnp.float32)
    part = jnp.dot(onehot, out2, preferred_element_type=jnp.float32)

    @pl.when(i == 0)
    def _():
        p2_acc[...] = jnp.zeros_like(p2_acc)

    p2_acc[...] += part

    @pl.when(i == NB - 1)
    def _():
        p = jnp.concatenate([p1_ref[...], p2_acc[...]], axis=1)
        h = jnp.dot(p, wl1_ref[...], preferred_element_type=jnp.float32)
        h = jax.nn.relu(h + bl1_ref[...])
        h = jnp.dot(h, wl2_ref[...], preferred_element_type=jnp.float32)
        h = h + bl2_ref[...]
        m = jnp.max(h, axis=1, keepdims=True)
        lse = jnp.log(jnp.sum(jnp.exp(h - m), axis=1, keepdims=True))
        h_ref[...] = h
        lsm_ref[...] = h - m - lse


def _tc_stage3(acc2, g2, deg2, b2r, batch3, p1, Wl1, bl1r, Wl2, bl2r):
    return pl.pallas_call(
        _tc3_body,
        grid=(NB,),
        in_specs=[
            pl.BlockSpec((NC, BN, D), lambda i: (0, i, 0)),
            pl.BlockSpec((BN, D), lambda i: (i, 0)),
            pl.BlockSpec((NC * NS, BN, 1), lambda i: (0, i, 0)),
            pl.BlockSpec((1, D), lambda i: (0, 0)),
            pl.BlockSpec((1, 1, BN), lambda i: (i, 0, 0)),
            pl.BlockSpec((G, D), lambda i: (0, 0)),
            pl.BlockSpec((2 * D, 2 * D), lambda i: (0, 0)),
            pl.BlockSpec((1, 2 * D), lambda i: (0, 0)),
            pl.BlockSpec((2 * D, 10), lambda i: (0, 0)),
            pl.BlockSpec((1, 10), lambda i: (0, 0)),
        ],
        out_specs=[
            pl.BlockSpec((G, 10), lambda i: (0, 0)),
            pl.BlockSpec((G, 10), lambda i: (0, 0)),
        ],
        out_shape=[
            jax.ShapeDtypeStruct((G, 10), jnp.float32),
            jax.ShapeDtypeStruct((G, 10), jnp.float32),
        ],
        scratch_shapes=[pltpu.VMEM((G, D), jnp.float32)],
    )(acc2, g2, deg2, b2r, batch3, p1, Wl1, bl1r, Wl2, bl2r)


def kernel(x, edge_index, batch, W1, b1, W2, b2, Wl1, bl1, Wl2, bl2):
    src = edge_index[0]
    dst = edge_index[1]
    batch3 = jnp.reshape(batch, (NB, 1, BN))

    srcp = jnp.pad(src, (0, EPAD - E))
    pad_dst = N + jnp.arange(EPAD - E, dtype=jnp.int32) % (NACC - N)
    dstp = jnp.concatenate([dst, pad_dst])

    deg2 = jnp.reshape(_sc_degree(dst), (NC * NS, NPAD, 1))
    g1 = _tc_stage1(x, W1, deg2)
    acc1 = _sc_scatter(g1, srcp, dstp)
    g2, p1 = _tc_stage2(acc1, g1, deg2, W2, jnp.reshape(b1, (1, D)), batch3)
    acc2 = _sc_scatter(g2, srcp, dstp)
    h, lsm = _tc_stage3(acc2, g2, deg2, jnp.reshape(b2, (1, D)), batch3, p1,
                        Wl1, jnp.reshape(bl1, (1, 2 * D)), Wl2,
                        jnp.reshape(bl2, (1, 10)))
    return (h, lsm)


# speedup vs baseline: 2.0299x; 2.0225x over previous
import functools

import jax
import jax.numpy as jnp
from jax import lax
from jax.experimental import pallas as pl
from jax.experimental.pallas import tpu as pltpu
from jax.experimental.pallas import tpu_sc as plsc

N = 10000
NPAD = 10240
E = 320000
D = 128
G = 64
NC = 2
NS = 16
EK = 80
NCHUNK = 125
WE = E // (NC * NS)
RPS = 640
RLAST = N - (NS - 1) * RPS
EDGES_PER_SUB = E // (NC * NS)
NB = 10
BN = N // NB


def _sc_mesh():
    return plsc.VectorSubcoreMesh(core_axis_name="c", subcore_axis_name="s")


def _row_slab(s, copy_fn):
    @pl.when(s < NS - 1)
    def _():
        copy_fn(s * RPS, RPS)

    @pl.when(s == NS - 1)
    def _():
        copy_fn((NS - 1) * RPS, RLAST)


def _deg_body(dst_hbm, out_hbm, didx_v, hist_v):
    c = lax.axis_index("c")
    s = lax.axis_index("s")
    zeros16 = jnp.zeros((16,), jnp.float32)

    def zstep(i, carry):
        hist_v[pl.ds(i * 16, 16)] = zeros16
        return carry

    lax.fori_loop(0, NPAD // 16, zstep, 0)
    base = c * (E // NC) + s * EDGES_PER_SUB
    pltpu.sync_copy(dst_hbm.at[pl.ds(base, EDGES_PER_SUB)], didx_v)
    ones16 = jnp.ones((16,), jnp.float32)

    def step(i, carry):
        idx = didx_v[pl.ds(i * 16, 16)]
        plsc.addupdate_scatter(hist_v, [idx], ones16)
        return carry

    lax.fori_loop(0, EDGES_PER_SUB // 16, step, 0)
    pltpu.sync_copy(hist_v, out_hbm.at[c, s])


def _sc_degree(dst):
    return pl.kernel(
        _deg_body,
        out_type=jax.ShapeDtypeStruct((NC, NS, NPAD), jnp.float32),
        mesh=_sc_mesh(),
        compiler_params=pltpu.CompilerParams(needs_layout_passes=False),
        scratch_types=[
            pltpu.VMEM((EDGES_PER_SUB,), jnp.int32),
            pltpu.VMEM((NPAD,), jnp.float32),
        ],
    )(dst)


def _scatter_body(g_hbm, src_hbm, dst_hbm, out_hbm, sidx0, sidx1, didx0,
                  didx1, rows0, rows1, gs0, gs1, ss0, ss1, acc_sh):
    c = lax.axis_index("c")
    s = lax.axis_index("s")
    w = c * NS + s
    _row_slab(s, lambda r0, nr: pltpu.sync_copy(
        g_hbm.at[pl.ds(r0, nr)], acc_sh.at[pl.ds(r0, nr)]))
    plsc.subcore_barrier()
    base = w * WE

    sidx = (sidx0, sidx1)
    didx = (didx0, didx1)
    rows = (rows0, rows1)
    gsem = (gs0, gs1)
    ssem = (ss0, ss1)

    def chunk(j, b):
        pltpu.sync_copy(src_hbm.at[pl.ds(base + j * EK, EK)], sidx[b])
        pltpu.sync_copy(dst_hbm.at[pl.ds(base + j * EK, EK)], didx[b])
        pltpu.async_copy(g_hbm.at[sidx[b]], rows[b], gsem[b]).wait()
        pltpu.async_copy(rows[b], acc_sh.at[didx[b]], ssem[b], add=True)

    def swait(b):
        pltpu.make_async_copy(rows[b], acc_sh.at[didx[b]], ssem[b]).wait()

    chunk(0, 0)
    chunk(1, 1)

    def pair(k, carry):
        for jj in (0, 1):
            j = 2 * k + jj + 2
            b = jj
            swait(b)
            chunk(j, b)
        return carry

    lax.fori_loop(0, (NCHUNK - 3) // 2, pair, 0)
    swait(0)
    chunk(NCHUNK - 1, 0)
    swait(0)
    swait(1)
    plsc.subcore_barrier()
    _row_slab(s, lambda r0, nr: pltpu.sync_copy(
        acc_sh.at[pl.ds(r0, nr)], out_hbm.at[c, pl.ds(r0, nr)]))


def _sc_scatter(g, srcp, dstp):
    return pl.kernel(
        _scatter_body,
        out_type=jax.ShapeDtypeStruct((NC, N, D), jnp.float32),
        mesh=_sc_mesh(),
        scratch_types=[
            pltpu.VMEM((EK,), jnp.int32),
            pltpu.VMEM((EK,), jnp.int32),
            pltpu.VMEM((EK,), jnp.int32),
            pltpu.VMEM((EK,), jnp.int32),
            pltpu.VMEM((EK, D), jnp.float32),
            pltpu.VMEM((EK, D), jnp.float32),
            pltpu.SemaphoreType.DMA,
            pltpu.SemaphoreType.DMA,
            pltpu.SemaphoreType.DMA,
            pltpu.SemaphoreType.DMA,
            pltpu.VMEM_SHARED((N, D), jnp.float32),
        ],
    )(g, srcp, dstp)


def _tc1_body(x_ref, w_ref, deg_ref, g_ref):
    deg = jnp.sum(deg_ref[...], axis=0) + 1.0
    dinv = lax.rsqrt(deg)
    h = jnp.dot(x_ref[...], w_ref[...], preferred_element_type=jnp.float32)
    g_ref[...] = h * dinv


def _tc_stage1(x, W1, deg2):
    return pl.pallas_call(
        _tc1_body,
        grid=(NB,),
        in_specs=[
            pl.BlockSpec((BN, D), lambda i: (i, 0)),
            pl.BlockSpec((D, D), lambda i: (0, 0)),
            pl.BlockSpec((NC * NS, BN, 1), lambda i: (0, i, 0)),
        ],
        out_specs=pl.BlockSpec((BN, D), lambda i: (i, 0)),
        out_shape=jax.ShapeDtypeStruct((N, D), jnp.float32),
    )(x, W1, deg2)


def _tc2_body(acc_ref, g1_ref, deg_ref, w2_ref, b1_ref, batch_ref,
              g2_ref, p1_ref, p1_acc):
    i = pl.program_id(0)
    deg = jnp.sum(deg_ref[...], axis=0) + 1.0
    dinv = lax.rsqrt(deg)
    esum = acc_ref[0] + acc_ref[1] - g1_ref[...]
    out1 = jax.nn.relu(esum * dinv + b1_ref[...])
    h2 = jnp.dot(out1, w2_ref[...], preferred_element_type=jnp.float32)
    g2_ref[...] = h2 * dinv
    onehot = (batch_ref[0] == lax.broadcasted_iota(jnp.int32, (G, BN), 0)
              ).astype(jnp.float32)
    part = jnp.dot(onehot, out1, preferred_element_type=jnp.float32)

    @pl.when(i == 0)
    def _():
        p1_acc[...] = jnp.zeros_like(p1_acc)

    p1_acc[...] += part

    @pl.when(i == NB - 1)
    def _():
        p1_ref[...] = p1_acc[...]


def _tc_stage2(acc1, g1, deg2, W2, b1r, batch3):
    return pl.pallas_call(
        _tc2_body,
        grid=(NB,),
        in_specs=[
            pl.BlockSpec((NC, BN, D), lambda i: (0, i, 0)),
            pl.BlockSpec((BN, D), lambda i: (i, 0)),
            pl.BlockSpec((NC * NS, BN, 1), lambda i: (0, i, 0)),
            pl.BlockSpec((D, D), lambda i: (0, 0)),
            pl.BlockSpec((1, D), lambda i: (0, 0)),
            pl.BlockSpec((1, 1, BN), lambda i: (i, 0, 0)),
        ],
        out_specs=[
            pl.BlockSpec((BN, D), lambda i: (i, 0)),
            pl.BlockSpec((G, D), lambda i: (0, 0)),
        ],
        out_shape=[
            jax.ShapeDtypeStruct((N, D), jnp.float32),
            jax.ShapeDtypeStruct((G, D), jnp.float32),
        ],
        scratch_shapes=[pltpu.VMEM((G, D), jnp.float32)],
    )(acc1, g1, deg2, W2, b1r, batch3)


def _tc3_body(acc_ref, g2_ref, deg_ref, b2_ref, batch_ref, p1_ref,
              wl1_ref, bl1_ref, wl2_ref, bl2_ref, h_ref, lsm_ref, p2_acc):
    i = pl.program_id(0)
    deg = jnp.sum(deg_ref[...], axis=0) + 1.0
    dinv = lax.rsqrt(deg)
    esum = acc_ref[0] + acc_ref[1] - g2_ref[...]
    out2 = jax.nn.relu(esum * dinv + b2_ref[...])
    onehot = (batch_ref[0] == lax.broadcasted_iota(jnp.int32, (G, BN), 0)
              ).astype(jnp.float32)
    part = jnp.dot(onehot, out2, preferred_element_type=jnp.float32)

    @pl.when(i == 0)
    def _():
        p2_acc[...] = jnp.zeros_like(p2_acc)

    p2_acc[...] += part

    @pl.when(i == NB - 1)
    def _():
        p = jnp.concatenate([p1_ref[...], p2_acc[...]], axis=1)
        h = jnp.dot(p, wl1_ref[...], preferred_element_type=jnp.float32)
        h = jax.nn.relu(h + bl1_ref[...])
        h = jnp.dot(h, wl2_ref[...], preferred_element_type=jnp.float32)
        h = h + bl2_ref[...]
        m = jnp.max(h, axis=1, keepdims=True)
        lse = jnp.log(jnp.sum(jnp.exp(h - m), axis=1, keepdims=True))
        h_ref[...] = h
        lsm_ref[...] = h - m - lse


def _tc_stage3(acc2, g2, deg2, b2r, batch3, p1, Wl1, bl1r, Wl2, bl2r):
    return pl.pallas_call(
        _tc3_body,
        grid=(NB,),
        in_specs=[
            pl.BlockSpec((NC, BN, D), lambda i: (0, i, 0)),
            pl.BlockSpec((BN, D), lambda i: (i, 0)),
            pl.BlockSpec((NC * NS, BN, 1), lambda i: (0, i, 0)),
            pl.BlockSpec((1, D), lambda i: (0, 0)),
            pl.BlockSpec((1, 1, BN), lambda i: (i, 0, 0)),
            pl.BlockSpec((G, D), lambda i: (0, 0)),
            pl.BlockSpec((2 * D, 2 * D), lambda i: (0, 0)),
            pl.BlockSpec((1, 2 * D), lambda i: (0, 0)),
            pl.BlockSpec((2 * D, 10), lambda i: (0, 0)),
            pl.BlockSpec((1, 10), lambda i: (0, 0)),
        ],
        out_specs=[
            pl.BlockSpec((G, 10), lambda i: (0, 0)),
            pl.BlockSpec((G, 10), lambda i: (0, 0)),
        ],
        out_shape=[
            jax.ShapeDtypeStruct((G, 10), jnp.float32),
            jax.ShapeDtypeStruct((G, 10), jnp.float32),
        ],
        scratch_shapes=[pltpu.VMEM((G, D), jnp.float32)],
    )(acc2, g2, deg2, b2r, batch3, p1, Wl1, bl1r, Wl2, bl2r)


def kernel(x, edge_index, batch, W1, b1, W2, b2, Wl1, bl1, Wl2, bl2):
    src = edge_index[0]
    dst = edge_index[1]
    batch3 = jnp.reshape(batch, (NB, 1, BN))

    srcp = src
    dstp = dst

    deg2 = jnp.reshape(_sc_degree(dst), (NC * NS, NPAD, 1))
    g1 = _tc_stage1(x, W1, deg2)
    acc1 = _sc_scatter(g1, srcp, dstp)
    g2, p1 = _tc_stage2(acc1, g1, deg2, W2, jnp.reshape(b1, (1, D)), batch3)
    acc2 = _sc_scatter(g2, srcp, dstp)
    h, lsm = _tc_stage3(acc2, g2, deg2, jnp.reshape(b2, (1, D)), batch3, p1,
                        Wl1, jnp.reshape(bl1, (1, 2 * D)), Wl2,
                        jnp.reshape(bl2, (1, 10)))
    return (h, lsm)


# speedup vs baseline: 3.0892x; 1.5218x over previous
import functools

import jax
import jax.numpy as jnp
from jax import lax
from jax.experimental import pallas as pl
from jax.experimental.pallas import tpu as pltpu
from jax.experimental.pallas import tpu_sc as plsc

N = 10000
NPAD = 10240
E = 320000
D = 128
G = 64
NC = 2
NS = 16
EK = 80
NCHUNK = 125
WE = E // (NC * NS)
RPS = 640
RLAST = N - (NS - 1) * RPS
EDGES_PER_SUB = E // (NC * NS)
NB = 10
BN = N // NB


def _sc_mesh():
    return plsc.VectorSubcoreMesh(core_axis_name="c", subcore_axis_name="s")


def _row_slab(s, copy_fn):
    @pl.when(s < NS - 1)
    def _():
        copy_fn(s * RPS, RPS)

    @pl.when(s == NS - 1)
    def _():
        copy_fn((NS - 1) * RPS, RLAST)


def _deg_body(dst_hbm, out_hbm, didx_v, hist_v):
    c = lax.axis_index("c")
    s = lax.axis_index("s")
    zeros16 = jnp.zeros((16,), jnp.float32)

    def zstep(i, carry):
        hist_v[pl.ds(i * 16, 16)] = zeros16
        return carry

    lax.fori_loop(0, NPAD // 16, zstep, 0)
    base = c * (E // NC) + s * EDGES_PER_SUB
    pltpu.sync_copy(dst_hbm.at[pl.ds(base, EDGES_PER_SUB)], didx_v)
    ones16 = jnp.ones((16,), jnp.float32)

    def step(i, carry):
        idx = didx_v[pl.ds(i * 16, 16)]
        plsc.addupdate_scatter(hist_v, [idx], ones16)
        return carry

    lax.fori_loop(0, EDGES_PER_SUB // 16, step, 0)
    pltpu.sync_copy(hist_v, out_hbm.at[c, s])


def _sc_degree(dst):
    return pl.kernel(
        _deg_body,
        out_type=jax.ShapeDtypeStruct((NC, NS, NPAD), jnp.float32),
        mesh=_sc_mesh(),
        compiler_params=pltpu.CompilerParams(needs_layout_passes=False),
        scratch_types=[
            pltpu.VMEM((EDGES_PER_SUB,), jnp.int32),
            pltpu.VMEM((NPAD,), jnp.float32),
        ],
    )(dst)


def _scatter_body(g_hbm, src_hbm, dst_hbm, out_hbm,
                  si0, si1, si2, si3, di0, di1, di2, di3,
                  r0_, r1_, r2_, r3_,
                  is0, is1, is2, is3, gs0, gs1, gs2, gs3,
                  ss0, ss1, ss2, ss3, acc_sh):
    c = lax.axis_index("c")
    s = lax.axis_index("s")
    w = c * NS + s
    _row_slab(s, lambda r0, nr: pltpu.sync_copy(
        g_hbm.at[pl.ds(r0, nr)], acc_sh.at[pl.ds(r0, nr)]))
    plsc.subcore_barrier()
    base = w * WE

    sidx = (si0, si1, si2, si3)
    didx = (di0, di1, di2, di3)
    rows = (r0_, r1_, r2_, r3_)
    isem = (is0, is1, is2, is3)
    gsem = (gs0, gs1, gs2, gs3)
    ssem = (ss0, ss1, ss2, ss3)

    def prefetch(j, b):
        pltpu.async_copy(src_hbm.at[pl.ds(base + j * EK, EK)], sidx[b],
                         isem[b])
        pltpu.async_copy(dst_hbm.at[pl.ds(base + j * EK, EK)], didx[b],
                         isem[b])

    def iwait(b):
        pltpu.make_async_copy(src_hbm.at[pl.ds(0, EK)], sidx[b],
                              isem[b]).wait()
        pltpu.make_async_copy(src_hbm.at[pl.ds(0, EK)], didx[b],
                              isem[b]).wait()

    def gwork(j, b):
        iwait(b)
        pltpu.async_copy(g_hbm.at[sidx[b]], rows[b], gsem[b]).wait()
        pltpu.async_copy(rows[b], acc_sh.at[didx[b]], ssem[b], add=True)

    def swait(b):
        pltpu.make_async_copy(rows[b], acc_sh.at[didx[b]], ssem[b]).wait()

    prefetch(0, 0)
    prefetch(1, 1)
    gwork(0, 0)
    prefetch(2, 2)
    gwork(1, 1)
    prefetch(3, 3)

    def quad(k, carry):
        for jj in range(4):
            j = 4 * k + jj + 2
            b = (jj + 2) % 4
            gwork(j, b)
            swait(jj)
            prefetch(j + 2, jj)
        return carry

    lax.fori_loop(0, 30, quad, 0)
    gwork(122, 2)
    swait(0)
    prefetch(124, 0)
    gwork(123, 3)
    swait(1)
    gwork(124, 0)
    swait(2)
    swait(3)
    swait(0)
    plsc.subcore_barrier()
    _row_slab(s, lambda r0, nr: pltpu.sync_copy(
        acc_sh.at[pl.ds(r0, nr)], out_hbm.at[c, pl.ds(r0, nr)]))


def _sc_scatter(g, srcp, dstp):
    idx_t = [pltpu.VMEM((EK,), jnp.int32) for _ in range(8)]
    row_t = [pltpu.VMEM((EK, D), jnp.float32) for _ in range(4)]
    sem_t = [pltpu.SemaphoreType.DMA for _ in range(12)]
    return pl.kernel(
        _scatter_body,
        out_type=jax.ShapeDtypeStruct((NC, N, D), jnp.float32),
        mesh=_sc_mesh(),
        scratch_types=idx_t + row_t + sem_t + [
            pltpu.VMEM_SHARED((N, D), jnp.float32),
        ],
    )(g, srcp, dstp)


def _tc1_body(x_ref, w_ref, deg_ref, g_ref):
    deg = jnp.sum(deg_ref[...], axis=0) + 1.0
    dinv = lax.rsqrt(deg)
    h = jnp.dot(x_ref[...], w_ref[...], preferred_element_type=jnp.float32)
    g_ref[...] = h * dinv


def _tc_stage1(x, W1, deg2):
    return pl.pallas_call(
        _tc1_body,
        grid=(NB,),
        in_specs=[
            pl.BlockSpec((BN, D), lambda i: (i, 0)),
            pl.BlockSpec((D, D), lambda i: (0, 0)),
            pl.BlockSpec((NC * NS, BN, 1), lambda i: (0, i, 0)),
        ],
        out_specs=pl.BlockSpec((BN, D), lambda i: (i, 0)),
        out_shape=jax.ShapeDtypeStruct((N, D), jnp.float32),
    )(x, W1, deg2)


def _tc2_body(acc_ref, g1_ref, deg_ref, w2_ref, b1_ref, batch_ref,
              g2_ref, p1_ref, p1_acc):
    i = pl.program_id(0)
    deg = jnp.sum(deg_ref[...], axis=0) + 1.0
    dinv = lax.rsqrt(deg)
    esum = acc_ref[0] + acc_ref[1] - g1_ref[...]
    out1 = jax.nn.relu(esum * dinv + b1_ref[...])
    h2 = jnp.dot(out1, w2_ref[...], preferred_element_type=jnp.float32)
    g2_ref[...] = h2 * dinv
    onehot = (batch_ref[0] == lax.broadcasted_iota(jnp.int32, (G, BN), 0)
              ).astype(jnp.float32)
    part = jnp.dot(onehot, out1, preferred_element_type=jnp.float32)

    @pl.when(i == 0)
    def _():
        p1_acc[...] = jnp.zeros_like(p1_acc)

    p1_acc[...] += part

    @pl.when(i == NB - 1)
    def _():
        p1_ref[...] = p1_acc[...]


def _tc_stage2(acc1, g1, deg2, W2, b1r, batch3):
    return pl.pallas_call(
        _tc2_body,
        grid=(NB,),
        in_specs=[
            pl.BlockSpec((NC, BN, D), lambda i: (0, i, 0)),
            pl.BlockSpec((BN, D), lambda i: (i, 0)),
            pl.BlockSpec((NC * NS, BN, 1), lambda i: (0, i, 0)),
            pl.BlockSpec((D, D), lambda i: (0, 0)),
            pl.BlockSpec((1, D), lambda i: (0, 0)),
            pl.BlockSpec((1, 1, BN), lambda i: (i, 0, 0)),
        ],
        out_specs=[
            pl.BlockSpec((BN, D), lambda i: (i, 0)),
            pl.BlockSpec((G, D), lambda i: (0, 0)),
        ],
        out_shape=[
            jax.ShapeDtypeStruct((N, D), jnp.float32),
            jax.ShapeDtypeStruct((G, D), jnp.float32),
        ],
        scratch_shapes=[pltpu.VMEM((G, D), jnp.float32)],
    )(acc1, g1, deg2, W2, b1r, batch3)


def _tc3_body(acc_ref, g2_ref, deg_ref, b2_ref, batch_ref, p1_ref,
              wl1_ref, bl1_ref, wl2_ref, bl2_ref, h_ref, lsm_ref, p2_acc):
    i = pl.program_id(0)
    deg = jnp.sum(deg_ref[...], axis=0) + 1.0
    dinv = lax.rsqrt(deg)
    esum = acc_ref[0] + acc_ref[1] - g2_ref[...]
    out2 = jax.nn.relu(esum * dinv + b2_ref[...])
    onehot = (batch_ref[0] == lax.broadcasted_iota(jnp.int32, (G, BN), 0)
              ).astype(jnp.float32)
    part = jnp.dot(onehot, out2, preferred_element_type=jnp.float32)

    @pl.when(i == 0)
    def _():
        p2_acc[...] = jnp.zeros_like(p2_acc)

    p2_acc[...] += part

    @pl.when(i == NB - 1)
    def _():
        p = jnp.concatenate([p1_ref[...], p2_acc[...]], axis=1)
        h = jnp.dot(p, wl1_ref[...], preferred_element_type=jnp.float32)
        h = jax.nn.relu(h + bl1_ref[...])
        h = jnp.dot(h, wl2_ref[...], preferred_element_type=jnp.float32)
        h = h + bl2_ref[...]
        m = jnp.max(h, axis=1, keepdims=True)
        lse = jnp.log(jnp.sum(jnp.exp(h - m), axis=1, keepdims=True))
        h_ref[...] = h
        lsm_ref[...] = h - m - lse


def _tc_stage3(acc2, g2, deg2, b2r, batch3, p1, Wl1, bl1r, Wl2, bl2r):
    return pl.pallas_call(
        _tc3_body,
        grid=(NB,),
        in_specs=[
            pl.BlockSpec((NC, BN, D), lambda i: (0, i, 0)),
            pl.BlockSpec((BN, D), lambda i: (i, 0)),
            pl.BlockSpec((NC * NS, BN, 1), lambda i: (0, i, 0)),
            pl.BlockSpec((1, D), lambda i: (0, 0)),
            pl.BlockSpec((1, 1, BN), lambda i: (i, 0, 0)),
            pl.BlockSpec((G, D), lambda i: (0, 0)),
            pl.BlockSpec((2 * D, 2 * D), lambda i: (0, 0)),
            pl.BlockSpec((1, 2 * D), lambda i: (0, 0)),
            pl.BlockSpec((2 * D, 10), lambda i: (0, 0)),
            pl.BlockSpec((1, 10), lambda i: (0, 0)),
        ],
        out_specs=[
            pl.BlockSpec((G, 10), lambda i: (0, 0)),
            pl.BlockSpec((G, 10), lambda i: (0, 0)),
        ],
        out_shape=[
            jax.ShapeDtypeStruct((G, 10), jnp.float32),
            jax.ShapeDtypeStruct((G, 10), jnp.float32),
        ],
        scratch_shapes=[pltpu.VMEM((G, D), jnp.float32)],
    )(acc2, g2, deg2, b2r, batch3, p1, Wl1, bl1r, Wl2, bl2r)


def kernel(x, edge_index, batch, W1, b1, W2, b2, Wl1, bl1, Wl2, bl2):
    src = edge_index[0]
    dst = edge_index[1]
    batch3 = jnp.reshape(batch, (NB, 1, BN))

    srcp = src
    dstp = dst

    deg2 = jnp.reshape(_sc_degree(dst), (NC * NS, NPAD, 1))
    g1 = _tc_stage1(x, W1, deg2)
    acc1 = _sc_scatter(g1, srcp, dstp)
    g2, p1 = _tc_stage2(acc1, g1, deg2, W2, jnp.reshape(b1, (1, D)), batch3)
    acc2 = _sc_scatter(g2, srcp, dstp)
    h, lsm = _tc_stage3(acc2, g2, deg2, jnp.reshape(b2, (1, D)), batch3, p1,
                        Wl1, jnp.reshape(bl1, (1, 2 * D)), Wl2,
                        jnp.reshape(bl2, (1, 10)))
    return (h, lsm)


# speedup vs baseline: 4.0808x; 1.3210x over previous
import functools

import jax
import jax.numpy as jnp
from jax import lax
from jax.experimental import pallas as pl
from jax.experimental.pallas import tpu as pltpu
from jax.experimental.pallas import tpu_sc as plsc

N = 10000
NPAD = 10240
E = 320000
D = 128
G = 64
NC = 2
NS = 16
EK = 80
NCHUNK = 125
WE = E // (NC * NS)
RPS = NPAD // NS
EDGES_PER_SUB = E // (NC * NS)
NB = 10
BN = NPAD // NB


def _sc_mesh():
    return plsc.VectorSubcoreMesh(core_axis_name="c", subcore_axis_name="s")


def _deg_body(dst_hbm, out_hbm, didx_v, hist_v):
    c = lax.axis_index("c")
    s = lax.axis_index("s")
    zeros16 = jnp.zeros((16,), jnp.float32)

    def zstep(i, carry):
        hist_v[pl.ds(i * 16, 16)] = zeros16
        return carry

    lax.fori_loop(0, NPAD // 16, zstep, 0)
    base = c * (E // NC) + s * EDGES_PER_SUB
    pltpu.sync_copy(dst_hbm.at[pl.ds(base, EDGES_PER_SUB)], didx_v)
    ones16 = jnp.ones((16,), jnp.float32)

    def step(i, carry):
        idx = didx_v[pl.ds(i * 16, 16)]
        plsc.addupdate_scatter(hist_v, [idx], ones16)
        return carry

    lax.fori_loop(0, EDGES_PER_SUB // 16, step, 0)
    pltpu.sync_copy(hist_v, out_hbm.at[c, s])


def _sc_degree(dst):
    return pl.kernel(
        _deg_body,
        out_type=jax.ShapeDtypeStruct((NC, NS, NPAD), jnp.float32),
        mesh=_sc_mesh(),
        compiler_params=pltpu.CompilerParams(needs_layout_passes=False),
        scratch_types=[
            pltpu.VMEM((EDGES_PER_SUB,), jnp.int32),
            pltpu.VMEM((NPAD,), jnp.float32),
        ],
    )(dst)


def _scatter_body(g_hbm, src_hbm, dst_hbm, out_hbm,
                  si0, si1, si2, si3, di0, di1, di2, di3,
                  r0_, r1_, r2_, r3_,
                  is0, is1, is2, is3, gs0, gs1, gs2, gs3,
                  ss0, ss1, ss2, ss3, acc_sh):
    c = lax.axis_index("c")
    s = lax.axis_index("s")
    w = c * NS + s
    r0 = s * RPS
    pltpu.sync_copy(g_hbm.at[pl.ds(r0, RPS)], acc_sh.at[pl.ds(r0, RPS)])
    plsc.subcore_barrier()
    base = w * WE

    sidx = (si0, si1, si2, si3)
    didx = (di0, di1, di2, di3)
    rows = (r0_, r1_, r2_, r3_)
    isem = (is0, is1, is2, is3)
    gsem = (gs0, gs1, gs2, gs3)
    ssem = (ss0, ss1, ss2, ss3)

    def prefetch(j, b):
        pltpu.async_copy(src_hbm.at[pl.ds(base + j * EK, EK)], sidx[b],
                         isem[b])
        pltpu.async_copy(dst_hbm.at[pl.ds(base + j * EK, EK)], didx[b],
                         isem[b])

    def iwait(b):
        pltpu.make_async_copy(src_hbm.at[pl.ds(0, EK)], sidx[b],
                              isem[b]).wait()
        pltpu.make_async_copy(src_hbm.at[pl.ds(0, EK)], didx[b],
                              isem[b]).wait()

    def gwork(j, b):
        iwait(b)
        pltpu.async_copy(g_hbm.at[sidx[b]], rows[b], gsem[b]).wait()
        pltpu.async_copy(rows[b], acc_sh.at[didx[b]], ssem[b], add=True)

    def swait(b):
        pltpu.make_async_copy(rows[b], acc_sh.at[didx[b]], ssem[b]).wait()

    prefetch(0, 0)
    prefetch(1, 1)
    gwork(0, 0)
    prefetch(2, 2)
    gwork(1, 1)
    prefetch(3, 3)

    def quad(k, carry):
        for jj in range(4):
            j = 4 * k + jj + 2
            b = (jj + 2) % 4
            gwork(j, b)
            swait(jj)
            prefetch(j + 2, jj)
        return carry

    lax.fori_loop(0, 30, quad, 0)
    gwork(122, 2)
    swait(0)
    prefetch(124, 0)
    gwork(123, 3)
    swait(1)
    gwork(124, 0)
    swait(2)
    swait(3)
    swait(0)
    plsc.subcore_barrier()
    pltpu.sync_copy(acc_sh.at[pl.ds(r0, RPS)], out_hbm.at[c, pl.ds(r0, RPS)])


def _sc_scatter(g, srcp, dstp):
    idx_t = [pltpu.VMEM((EK,), jnp.int32) for _ in range(8)]
    row_t = [pltpu.VMEM((EK, D), jnp.float32) for _ in range(4)]
    sem_t = [pltpu.SemaphoreType.DMA for _ in range(12)]
    return pl.kernel(
        _scatter_body,
        out_type=jax.ShapeDtypeStruct((NC, NPAD, D), jnp.float32),
        mesh=_sc_mesh(),
        scratch_types=idx_t + row_t + sem_t + [
            pltpu.VMEM_SHARED((NPAD, D), jnp.float32),
        ],
    )(g, srcp, dstp)


def _tc0_body(p_ref, dinv_ref):
    ones = jnp.ones((NC * NS, 1), jnp.float32)
    deg = lax.dot_general(p_ref[...], ones, (((0,), (0,)), ((), ()))) + 1.0
    dinv_ref[...] = jnp.broadcast_to(lax.rsqrt(deg), (BN, 8))


def _tc_stage0(parts):
    return pl.pallas_call(
        _tc0_body,
        grid=(NB,),
        in_specs=[pl.BlockSpec((NC * NS, BN), lambda i: (0, i))],
        out_specs=pl.BlockSpec((BN, 8), lambda i: (i, 0)),
        out_shape=jax.ShapeDtypeStruct((NPAD, 8), jnp.float32),
    )(parts)


def _tc1_body(x_ref, w_ref, dinv_ref, g_ref):
    dinv = dinv_ref[:, 0:1]
    h = jnp.dot(x_ref[...], w_ref[...], preferred_element_type=jnp.float32)
    g_ref[...] = h * dinv


def _tc_stage1(x, W1, dinv8):
    return pl.pallas_call(
        _tc1_body,
        grid=(NB,),
        in_specs=[
            pl.BlockSpec((BN, D), lambda i: (i, 0)),
            pl.BlockSpec((D, D), lambda i: (0, 0)),
            pl.BlockSpec((BN, 8), lambda i: (i, 0)),
        ],
        out_specs=pl.BlockSpec((BN, D), lambda i: (i, 0)),
        out_shape=jax.ShapeDtypeStruct((NPAD, D), jnp.float32),
    )(x, W1, dinv8)


def _tc2_body(acc_ref, g1_ref, dinv_ref, w2_ref, b1_ref, batch_ref,
              g2_ref, p1_ref, p1_acc):
    i = pl.program_id(0)
    dinv = dinv_ref[:, 0:1]
    esum = acc_ref[0] + acc_ref[1] - g1_ref[...]
    out1 = jax.nn.relu(esum * dinv + b1_ref[...])
    h2 = jnp.dot(out1, w2_ref[...], preferred_element_type=jnp.float32)
    g2_ref[...] = h2 * dinv
    onehot = (batch_ref[0] == lax.broadcasted_iota(jnp.int32, (G, BN), 0)
              ).astype(jnp.float32)
    part = jnp.dot(onehot, out1, preferred_element_type=jnp.float32)

    @pl.when(i == 0)
    def _():
        p1_acc[...] = jnp.zeros_like(p1_acc)

    p1_acc[...] += part

    @pl.when(i == NB - 1)
    def _():
        p1_ref[...] = p1_acc[...]


def _tc_stage2(acc1, g1, dinv8, W2, b1r, batch3):
    return pl.pallas_call(
        _tc2_body,
        grid=(NB,),
        in_specs=[
            pl.BlockSpec((NC, BN, D), lambda i: (0, i, 0)),
            pl.BlockSpec((BN, D), lambda i: (i, 0)),
            pl.BlockSpec((BN, 8), lambda i: (i, 0)),
            pl.BlockSpec((D, D), lambda i: (0, 0)),
            pl.BlockSpec((1, D), lambda i: (0, 0)),
            pl.BlockSpec((1, 1, BN), lambda i: (i, 0, 0)),
        ],
        out_specs=[
            pl.BlockSpec((BN, D), lambda i: (i, 0)),
            pl.BlockSpec((G, D), lambda i: (0, 0)),
        ],
        out_shape=[
            jax.ShapeDtypeStruct((NPAD, D), jnp.float32),
            jax.ShapeDtypeStruct((G, D), jnp.float32),
        ],
        scratch_shapes=[pltpu.VMEM((G, D), jnp.float32)],
    )(acc1, g1, dinv8, W2, b1r, batch3)


def _tc3_body(acc_ref, g2_ref, dinv_ref, b2_ref, batch_ref, p1_ref,
              wl1_ref, bl1_ref, wl2_ref, bl2_ref, h_ref, lsm_ref, p2_acc):
    i = pl.program_id(0)
    dinv = dinv_ref[:, 0:1]
    esum = acc_ref[0] + acc_ref[1] - g2_ref[...]
    out2 = jax.nn.relu(esum * dinv + b2_ref[...])
    onehot = (batch_ref[0] == lax.broadcasted_iota(jnp.int32, (G, BN), 0)
              ).astype(jnp.float32)
    part = jnp.dot(onehot, out2, preferred_element_type=jnp.float32)

    @pl.when(i == 0)
    def _():
        p2_acc[...] = jnp.zeros_like(p2_acc)

    p2_acc[...] += part

    @pl.when(i == NB - 1)
    def _():
        p = jnp.concatenate([p1_ref[...], p2_acc[...]], axis=1)
        h = jnp.dot(p, wl1_ref[...], preferred_element_type=jnp.float32)
        h = jax.nn.relu(h + bl1_ref[...])
        h = jnp.dot(h, wl2_ref[...], preferred_element_type=jnp.float32)
        h = h + bl2_ref[...]
        m = jnp.max(h, axis=1, keepdims=True)
        lse = jnp.log(jnp.sum(jnp.exp(h - m), axis=1, keepdims=True))
        h_ref[...] = h
        lsm_ref[...] = h - m - lse


def _tc_stage3(acc2, g2, dinv8, b2r, batch3, p1, Wl1, bl1r, Wl2, bl2r):
    return pl.pallas_call(
        _tc3_body,
        grid=(NB,),
        in_specs=[
            pl.BlockSpec((NC, BN, D), lambda i: (0, i, 0)),
            pl.BlockSpec((BN, D), lambda i: (i, 0)),
            pl.BlockSpec((BN, 8), lambda i: (i, 0)),
            pl.BlockSpec((1, D), lambda i: (0, 0)),
            pl.BlockSpec((1, 1, BN), lambda i: (i, 0, 0)),
            pl.BlockSpec((G, D), lambda i: (0, 0)),
            pl.BlockSpec((2 * D, 2 * D), lambda i: (0, 0)),
            pl.BlockSpec((1, 2 * D), lambda i: (0, 0)),
            pl.BlockSpec((2 * D, 10), lambda i: (0, 0)),
            pl.BlockSpec((1, 10), lambda i: (0, 0)),
        ],
        out_specs=[
            pl.BlockSpec((G, 10), lambda i: (0, 0)),
            pl.BlockSpec((G, 10), lambda i: (0, 0)),
        ],
        out_shape=[
            jax.ShapeDtypeStruct((G, 10), jnp.float32),
            jax.ShapeDtypeStruct((G, 10), jnp.float32),
        ],
        scratch_shapes=[pltpu.VMEM((G, D), jnp.float32)],
    )(acc2, g2, dinv8, b2r, batch3, p1, Wl1, bl1r, Wl2, bl2r)


def kernel(x, edge_index, batch, W1, b1, W2, b2, Wl1, bl1, Wl2, bl2):
    src = edge_index[0]
    dst = edge_index[1]
    batchp = jnp.pad(batch, (0, NPAD - N), constant_values=G)
    batch3 = jnp.reshape(batchp, (NB, 1, BN))

    srcp = src
    dstp = dst
    xp = jnp.pad(x, ((0, NPAD - N), (0, 0)))

    parts = jnp.reshape(_sc_degree(dst), (NC * NS, NPAD))
    dinv8 = _tc_stage0(parts)
    g1 = _tc_stage1(xp, W1, dinv8)
    acc1 = _sc_scatter(g1, srcp, dstp)
    g2, p1 = _tc_stage2(acc1, g1, dinv8, W2, jnp.reshape(b1, (1, D)), batch3)
    acc2 = _sc_scatter(g2, srcp, dstp)
    h, lsm = _tc_stage3(acc2, g2, dinv8, jnp.reshape(b2, (1, D)), batch3, p1,
                        Wl1, jnp.reshape(bl1, (1, 2 * D)), Wl2,
                        jnp.reshape(bl2, (1, 10)))
    return (h, lsm)


# speedup vs baseline: 4.1971x; 1.0285x over previous
import functools

import jax
import jax.numpy as jnp
from jax import lax
from jax.experimental import pallas as pl
from jax.experimental.pallas import tpu as pltpu
from jax.experimental.pallas import tpu_sc as plsc

N = 10000
NPAD = 10240
E = 320000
D = 128
G = 64
NC = 2
NS = 16
EK = 80
NCHUNK = 125
WE = E // (NC * NS)
RPS = NPAD // NS
EDGES_PER_SUB = E // (NC * NS)
NB = 10
BN = NPAD // NB


def _sc_mesh():
    return plsc.VectorSubcoreMesh(core_axis_name="c", subcore_axis_name="s")


def _deg_body(ei_hbm, out_hbm, didx_v, hist_v):
    c = lax.axis_index("c")
    s = lax.axis_index("s")
    zeros16 = jnp.zeros((16,), jnp.float32)

    def zstep(i, carry):
        hist_v[pl.ds(i * 16, 16)] = zeros16
        return carry

    lax.fori_loop(0, NPAD // 16, zstep, 0)
    base = c * (E // NC) + s * EDGES_PER_SUB
    pltpu.sync_copy(ei_hbm.at[pl.ds(E + base, EDGES_PER_SUB)], didx_v)
    ones16 = jnp.ones((16,), jnp.float32)

    def step(i, carry):
        idx = didx_v[pl.ds(i * 16, 16)]
        plsc.addupdate_scatter(hist_v, [idx], ones16)
        return carry

    lax.fori_loop(0, EDGES_PER_SUB // 16, step, 0)
    pltpu.sync_copy(hist_v, out_hbm.at[c, s])


def _sc_degree(ei):
    return pl.kernel(
        _deg_body,
        out_type=jax.ShapeDtypeStruct((NC, NS, NPAD), jnp.float32),
        mesh=_sc_mesh(),
        compiler_params=pltpu.CompilerParams(needs_layout_passes=False),
        scratch_types=[
            pltpu.VMEM((EDGES_PER_SUB,), jnp.int32),
            pltpu.VMEM((NPAD,), jnp.float32),
        ],
    )(ei)


def _scatter_body(g_hbm, ei_hbm, out_hbm,
                  si0, si1, si2, si3, di0, di1, di2, di3,
                  r0_, r1_, r2_, r3_,
                  is0, is1, is2, is3, gs0, gs1, gs2, gs3,
                  ss0, ss1, ss2, ss3, acc_sh):
    c = lax.axis_index("c")
    s = lax.axis_index("s")
    w = c * NS + s
    r0 = s * RPS
    pltpu.sync_copy(g_hbm.at[pl.ds(r0, RPS)], acc_sh.at[pl.ds(r0, RPS)])
    plsc.subcore_barrier()
    base = w * WE

    sidx = (si0, si1, si2, si3)
    didx = (di0, di1, di2, di3)
    rows = (r0_, r1_, r2_, r3_)
    isem = (is0, is1, is2, is3)
    gsem = (gs0, gs1, gs2, gs3)
    ssem = (ss0, ss1, ss2, ss3)

    def prefetch(j, b):
        pltpu.async_copy(ei_hbm.at[pl.ds(base + j * EK, EK)], sidx[b],
                         isem[b])
        pltpu.async_copy(ei_hbm.at[pl.ds(E + base + j * EK, EK)], didx[b],
                         isem[b])

    def iwait(b):
        pltpu.make_async_copy(ei_hbm.at[pl.ds(0, EK)], sidx[b],
                              isem[b]).wait()
        pltpu.make_async_copy(ei_hbm.at[pl.ds(0, EK)], didx[b],
                              isem[b]).wait()

    def gwork(j, b):
        iwait(b)
        pltpu.async_copy(g_hbm.at[sidx[b]], rows[b], gsem[b]).wait()
        pltpu.async_copy(rows[b], acc_sh.at[didx[b]], ssem[b], add=True)

    def swait(b):
        pltpu.make_async_copy(rows[b], acc_sh.at[didx[b]], ssem[b]).wait()

    prefetch(0, 0)
    prefetch(1, 1)
    gwork(0, 0)
    prefetch(2, 2)
    gwork(1, 1)
    prefetch(3, 3)

    def quad(k, carry):
        for jj in range(4):
            j = 4 * k + jj + 2
            b = (jj + 2) % 4
            gwork(j, b)
            swait(jj)
            prefetch(j + 2, jj)
        return carry

    lax.fori_loop(0, 30, quad, 0)
    gwork(122, 2)
    swait(0)
    prefetch(124, 0)
    gwork(123, 3)
    swait(1)
    gwork(124, 0)
    swait(2)
    swait(3)
    swait(0)
    plsc.subcore_barrier()
    pltpu.sync_copy(acc_sh.at[pl.ds(r0, RPS)], out_hbm.at[c, pl.ds(r0, RPS)])


def _sc_scatter(g, ei):
    idx_t = [pltpu.VMEM((EK,), jnp.int32) for _ in range(8)]
    row_t = [pltpu.VMEM((EK, D), jnp.float32) for _ in range(4)]
    sem_t = [pltpu.SemaphoreType.DMA for _ in range(12)]
    return pl.kernel(
        _scatter_body,
        out_type=jax.ShapeDtypeStruct((NC, NPAD, D), jnp.float32),
        mesh=_sc_mesh(),
        scratch_types=idx_t + row_t + sem_t + [
            pltpu.VMEM_SHARED((NPAD, D), jnp.float32),
        ],
    )(g, ei)


def _tc0_body(p_ref, dinv_ref):
    ones = jnp.ones((NC * NS, 1), jnp.float32)
    deg = lax.dot_general(p_ref[...], ones, (((0,), (0,)), ((), ()))) + 1.0
    dinv_ref[...] = jnp.broadcast_to(lax.rsqrt(deg), (BN, 8))


def _tc_stage0(parts):
    return pl.pallas_call(
        _tc0_body,
        grid=(NB,),
        in_specs=[pl.BlockSpec((NC * NS, BN), lambda i: (0, i))],
        out_specs=pl.BlockSpec((BN, 8), lambda i: (i, 0)),
        out_shape=jax.ShapeDtypeStruct((NPAD, 8), jnp.float32),
    )(parts)


def _tc1_body(x_ref, w_ref, dinv_ref, g_ref):
    dinv = dinv_ref[:, 0:1]
    h = jnp.dot(x_ref[...], w_ref[...], preferred_element_type=jnp.float32)
    g_ref[...] = h * dinv


def _tc_stage1(x, W1, dinv8):
    return pl.pallas_call(
        _tc1_body,
        grid=(NB,),
        in_specs=[
            pl.BlockSpec((BN, D), lambda i: (i, 0)),
            pl.BlockSpec((D, D), lambda i: (0, 0)),
            pl.BlockSpec((BN, 8), lambda i: (i, 0)),
        ],
        out_specs=pl.BlockSpec((BN, D), lambda i: (i, 0)),
        out_shape=jax.ShapeDtypeStruct((NPAD, D), jnp.float32),
    )(x, W1, dinv8)


def _tc2_body(acc_ref, g1_ref, dinv_ref, w2_ref, b1_ref, batch_ref,
              g2_ref, p1_ref, p1_acc):
    i = pl.program_id(0)
    dinv = dinv_ref[:, 0:1]
    esum = acc_ref[0] + acc_ref[1] - g1_ref[...]
    out1 = jax.nn.relu(esum * dinv + b1_ref[...])
    h2 = jnp.dot(out1, w2_ref[...], preferred_element_type=jnp.float32)
    g2_ref[...] = h2 * dinv
    onehot = (batch_ref[0] == lax.broadcasted_iota(jnp.int32, (G, BN), 0)
              ).astype(jnp.float32)
    part = jnp.dot(onehot, out1, preferred_element_type=jnp.float32)

    @pl.when(i == 0)
    def _():
        p1_acc[...] = jnp.zeros_like(p1_acc)

    p1_acc[...] += part

    @pl.when(i == NB - 1)
    def _():
        p1_ref[...] = p1_acc[...]


def _tc_stage2(acc1, g1, dinv8, W2, b1r, batch3):
    return pl.pallas_call(
        _tc2_body,
        grid=(NB,),
        in_specs=[
            pl.BlockSpec((NC, BN, D), lambda i: (0, i, 0)),
            pl.BlockSpec((BN, D), lambda i: (i, 0)),
            pl.BlockSpec((BN, 8), lambda i: (i, 0)),
            pl.BlockSpec((D, D), lambda i: (0, 0)),
            pl.BlockSpec((1, D), lambda i: (0, 0)),
            pl.BlockSpec((1, 1, BN), lambda i: (i, 0, 0)),
        ],
        out_specs=[
            pl.BlockSpec((BN, D), lambda i: (i, 0)),
            pl.BlockSpec((G, D), lambda i: (0, 0)),
        ],
        out_shape=[
            jax.ShapeDtypeStruct((NPAD, D), jnp.float32),
            jax.ShapeDtypeStruct((G, D), jnp.float32),
        ],
        scratch_shapes=[pltpu.VMEM((G, D), jnp.float32)],
    )(acc1, g1, dinv8, W2, b1r, batch3)


def _tc3_body(acc_ref, g2_ref, dinv_ref, b2_ref, batch_ref, p1_ref,
              wl1_ref, bl1_ref, wl2_ref, bl2_ref, h_ref, lsm_ref, p2_acc):
    i = pl.program_id(0)
    dinv = dinv_ref[:, 0:1]
    esum = acc_ref[0] + acc_ref[1] - g2_ref[...]
    out2 = jax.nn.relu(esum * dinv + b2_ref[...])
    onehot = (batch_ref[0] == lax.broadcasted_iota(jnp.int32, (G, BN), 0)
              ).astype(jnp.float32)
    part = jnp.dot(onehot, out2, preferred_element_type=jnp.float32)

    @pl.when(i == 0)
    def _():
        p2_acc[...] = jnp.zeros_like(p2_acc)

    p2_acc[...] += part

    @pl.when(i == NB - 1)
    def _():
        p = jnp.concatenate([p1_ref[...], p2_acc[...]], axis=1)
        h = jnp.dot(p, wl1_ref[...], preferred_element_type=jnp.float32)
        h = jax.nn.relu(h + bl1_ref[...])
        h = jnp.dot(h, wl2_ref[...], preferred_element_type=jnp.float32)
        h = h + bl2_ref[...]
        m = jnp.max(h, axis=1, keepdims=True)
        lse = jnp.log(jnp.sum(jnp.exp(h - m), axis=1, keepdims=True))
        h_ref[...] = h
        lsm_ref[...] = h - m - lse


def _tc_stage3(acc2, g2, dinv8, b2r, batch3, p1, Wl1, bl1r, Wl2, bl2r):
    return pl.pallas_call(
        _tc3_body,
        grid=(NB,),
        in_specs=[
            pl.BlockSpec((NC, BN, D), lambda i: (0, i, 0)),
            pl.BlockSpec((BN, D), lambda i: (i, 0)),
            pl.BlockSpec((BN, 8), lambda i: (i, 0)),
            pl.BlockSpec((1, D), lambda i: (0, 0)),
            pl.BlockSpec((1, 1, BN), lambda i: (i, 0, 0)),
            pl.BlockSpec((G, D), lambda i: (0, 0)),
            pl.BlockSpec((2 * D, 2 * D), lambda i: (0, 0)),
            pl.BlockSpec((1, 2 * D), lambda i: (0, 0)),
            pl.BlockSpec((2 * D, 10), lambda i: (0, 0)),
            pl.BlockSpec((1, 10), lambda i: (0, 0)),
        ],
        out_specs=[
            pl.BlockSpec((G, 10), lambda i: (0, 0)),
            pl.BlockSpec((G, 10), lambda i: (0, 0)),
        ],
        out_shape=[
            jax.ShapeDtypeStruct((G, 10), jnp.float32),
            jax.ShapeDtypeStruct((G, 10), jnp.float32),
        ],
        scratch_shapes=[pltpu.VMEM((G, D), jnp.float32)],
    )(acc2, g2, dinv8, b2r, batch3, p1, Wl1, bl1r, Wl2, bl2r)


def kernel(x, edge_index, batch, W1, b1, W2, b2, Wl1, bl1, Wl2, bl2):
    batchp = jnp.pad(batch, (0, NPAD - N), constant_values=G)
    batch3 = jnp.reshape(batchp, (NB, 1, BN))

    xp = jnp.pad(x, ((0, NPAD - N), (0, 0)))

    eif = jnp.reshape(edge_index, (2 * E,))
    parts = jnp.reshape(_sc_degree(eif), (NC * NS, NPAD))
    dinv8 = _tc_stage0(parts)
    g1 = _tc_stage1(xp, W1, dinv8)
    acc1 = _sc_scatter(g1, eif)
    g2, p1 = _tc_stage2(acc1, g1, dinv8, W2, jnp.reshape(b1, (1, D)), batch3)
    acc2 = _sc_scatter(g2, eif)
    h, lsm = _tc_stage3(acc2, g2, dinv8, jnp.reshape(b2, (1, D)), batch3, p1,
                        Wl1, jnp.reshape(bl1, (1, 2 * D)), Wl2,
                        jnp.reshape(bl2, (1, 10)))
    return (h, lsm)


# speedup vs baseline: 5.0495x; 1.2031x over previous
import functools

import jax
import jax.numpy as jnp
from jax import lax
from jax.experimental import pallas as pl
from jax.experimental.pallas import tpu as pltpu
from jax.experimental.pallas import tpu_sc as plsc

N = 10000
NPAD = 10240
E = 320000
D = 128
G = 64
NC = 2
NS = 16
EK = 80
NCHUNK = 125
WE = E // (NC * NS)
RPS = NPAD // NS
EDGES_PER_SUB = E // (NC * NS)
NB = 10
BN = NPAD // NB


def _sc_mesh():
    return plsc.VectorSubcoreMesh(core_axis_name="c", subcore_axis_name="s")


def _deg_body(ei_hbm, out_hbm, didx_v, hist_v):
    c = lax.axis_index("c")
    s = lax.axis_index("s")
    zeros16 = jnp.zeros((16,), jnp.float32)

    def zstep(i, carry):
        hist_v[pl.ds(i * 16, 16)] = zeros16
        return carry

    lax.fori_loop(0, NPAD // 16, zstep, 0)
    base = c * (E // NC) + s * EDGES_PER_SUB
    pltpu.sync_copy(ei_hbm.at[pl.ds(E + base, EDGES_PER_SUB)], didx_v)
    ones16 = jnp.ones((16,), jnp.float32)

    def step(i, carry):
        idx = didx_v[pl.ds(i * 16, 16)]
        plsc.addupdate_scatter(hist_v, [idx], ones16)
        return carry

    lax.fori_loop(0, EDGES_PER_SUB // 16, step, 0)
    pltpu.sync_copy(hist_v, out_hbm.at[c, s])


def _sc_degree(ei):
    return pl.kernel(
        _deg_body,
        out_type=jax.ShapeDtypeStruct((NC, NS, NPAD), jnp.float32),
        mesh=_sc_mesh(),
        compiler_params=pltpu.CompilerParams(needs_layout_passes=False),
        scratch_types=[
            pltpu.VMEM((EDGES_PER_SUB,), jnp.int32),
            pltpu.VMEM((NPAD,), jnp.float32),
        ],
    )(ei)


def _scatter_body(g_hbm, ei_hbm, out_hbm,
                  si0, si1, si2, si3, di0, di1, di2, di3,
                  r0_, r1_, r2_, r3_,
                  is0, is1, is2, is3, gs0, gs1, gs2, gs3,
                  ss0, ss1, ss2, ss3, acc_sh):
    c = lax.axis_index("c")
    s = lax.axis_index("s")
    w = c * NS + s
    r0 = s * RPS
    pltpu.sync_copy(g_hbm.at[pl.ds(r0, RPS)], acc_sh.at[pl.ds(r0, RPS)])
    plsc.subcore_barrier()
    base = w * WE

    sidx = (si0, si1, si2, si3)
    didx = (di0, di1, di2, di3)
    rows = (r0_, r1_, r2_, r3_)
    isem = (is0, is1, is2, is3)
    gsem = (gs0, gs1, gs2, gs3)
    ssem = (ss0, ss1, ss2, ss3)

    def prefetch(j, b):
        pltpu.async_copy(ei_hbm.at[pl.ds(base + j * EK, EK)], sidx[b],
                         isem[b])
        pltpu.async_copy(ei_hbm.at[pl.ds(E + base + j * EK, EK)], didx[b],
                         isem[b])

    def iwait(b):
        pltpu.make_async_copy(ei_hbm.at[pl.ds(0, EK)], sidx[b],
                              isem[b]).wait()
        pltpu.make_async_copy(ei_hbm.at[pl.ds(0, EK)], didx[b],
                              isem[b]).wait()

    def gissue(j, b):
        pltpu.async_copy(g_hbm.at[sidx[b]], rows[b], gsem[b])

    def gwait(b):
        pltpu.make_async_copy(g_hbm.at[sidx[b]], rows[b], gsem[b]).wait()

    def sissue(j, b):
        pltpu.async_copy(rows[b], acc_sh.at[didx[b]], ssem[b], add=True)

    def swait(b):
        pltpu.make_async_copy(rows[b], acc_sh.at[didx[b]], ssem[b]).wait()

    prefetch(0, 0)
    prefetch(1, 1)
    iwait(0)
    gissue(0, 0)
    iwait(1)
    gissue(1, 1)
    gwait(0)
    sissue(0, 0)
    prefetch(2, 2)
    iwait(2)
    gissue(2, 2)
    gwait(1)
    sissue(1, 1)
    prefetch(3, 3)

    def quad(k, carry):
        for jj in range(4):
            j = 4 * k + jj + 2
            b = (jj + 2) % 4
            swait((jj) % 4)
            iwait((jj + 3) % 4)
            gissue(j + 1, (jj + 3) % 4)
            gwait(b)
            sissue(j, b)
            prefetch(j + 2, jj % 4)
        return carry

    lax.fori_loop(0, 30, quad, 0)
    swait(0)
    iwait(3)
    gissue(123, 3)
    gwait(2)
    sissue(122, 2)
    prefetch(124, 0)
    swait(1)
    iwait(0)
    gissue(124, 0)
    gwait(3)
    sissue(123, 3)
    swait(2)
    gwait(0)
    sissue(124, 0)
    swait(3)
    swait(0)
    plsc.subcore_barrier()
    pltpu.sync_copy(acc_sh.at[pl.ds(r0, RPS)], out_hbm.at[c, pl.ds(r0, RPS)])


def _sc_scatter(g, ei):
    idx_t = [pltpu.VMEM((EK,), jnp.int32) for _ in range(8)]
    row_t = [pltpu.VMEM((EK, D), jnp.float32) for _ in range(4)]
    sem_t = [pltpu.SemaphoreType.DMA for _ in range(12)]
    return pl.kernel(
        _scatter_body,
        out_type=jax.ShapeDtypeStruct((NC, NPAD, D), jnp.float32),
        mesh=_sc_mesh(),
        scratch_types=idx_t + row_t + sem_t + [
            pltpu.VMEM_SHARED((NPAD, D), jnp.float32),
        ],
    )(g, ei)


def _tc0_body(p_ref, dinv_ref):
    ones = jnp.ones((NC * NS, 1), jnp.float32)
    deg = lax.dot_general(p_ref[...], ones, (((0,), (0,)), ((), ()))) + 1.0
    dinv_ref[...] = jnp.broadcast_to(lax.rsqrt(deg), (BN, 8))


def _tc_stage0(parts):
    return pl.pallas_call(
        _tc0_body,
        grid=(NB,),
        in_specs=[pl.BlockSpec((NC * NS, BN), lambda i: (0, i))],
        out_specs=pl.BlockSpec((BN, 8), lambda i: (i, 0)),
        out_shape=jax.ShapeDtypeStruct((NPAD, 8), jnp.float32),
    )(parts)


def _tc1_body(x_ref, w_ref, dinv_ref, g_ref):
    dinv = dinv_ref[:, 0:1]
    h = jnp.dot(x_ref[...], w_ref[...], preferred_element_type=jnp.float32)
    g_ref[...] = h * dinv


def _tc_stage1(x, W1, dinv8):
    return pl.pallas_call(
        _tc1_body,
        grid=(NB,),
        in_specs=[
            pl.BlockSpec((BN, D), lambda i: (i, 0)),
            pl.BlockSpec((D, D), lambda i: (0, 0)),
            pl.BlockSpec((BN, 8), lambda i: (i, 0)),
        ],
        out_specs=pl.BlockSpec((BN, D), lambda i: (i, 0)),
        out_shape=jax.ShapeDtypeStruct((NPAD, D), jnp.float32),
    )(x, W1, dinv8)


def _tc2_body(acc_ref, g1_ref, dinv_ref, w2_ref, b1_ref, batch_ref,
              g2_ref, p1_ref, p1_acc):
    i = pl.program_id(0)
    dinv = dinv_ref[:, 0:1]
    esum = acc_ref[0] + acc_ref[1] - g1_ref[...]
    out1 = jax.nn.relu(esum * dinv + b1_ref[...])
    h2 = jnp.dot(out1, w2_ref[...], preferred_element_type=jnp.float32)
    g2_ref[...] = h2 * dinv
    onehot = (batch_ref[0] == lax.broadcasted_iota(jnp.int32, (G, BN), 0)
              ).astype(jnp.float32)
    part = jnp.dot(onehot, out1, preferred_element_type=jnp.float32)

    @pl.when(i == 0)
    def _():
        p1_acc[...] = jnp.zeros_like(p1_acc)

    p1_acc[...] += part

    @pl.when(i == NB - 1)
    def _():
        p1_ref[...] = p1_acc[...]


def _tc_stage2(acc1, g1, dinv8, W2, b1r, batch3):
    return pl.pallas_call(
        _tc2_body,
        grid=(NB,),
        in_specs=[
            pl.BlockSpec((NC, BN, D), lambda i: (0, i, 0)),
            pl.BlockSpec((BN, D), lambda i: (i, 0)),
            pl.BlockSpec((BN, 8), lambda i: (i, 0)),
            pl.BlockSpec((D, D), lambda i: (0, 0)),
            pl.BlockSpec((1, D), lambda i: (0, 0)),
            pl.BlockSpec((1, 1, BN), lambda i: (i, 0, 0)),
        ],
        out_specs=[
            pl.BlockSpec((BN, D), lambda i: (i, 0)),
            pl.BlockSpec((G, D), lambda i: (0, 0)),
        ],
        out_shape=[
            jax.ShapeDtypeStruct((NPAD, D), jnp.float32),
            jax.ShapeDtypeStruct((G, D), jnp.float32),
        ],
        scratch_shapes=[pltpu.VMEM((G, D), jnp.float32)],
    )(acc1, g1, dinv8, W2, b1r, batch3)


def _tc3_body(acc_ref, g2_ref, dinv_ref, b2_ref, batch_ref, p1_ref,
              wl1_ref, bl1_ref, wl2_ref, bl2_ref, h_ref, lsm_ref, p2_acc):
    i = pl.program_id(0)
    dinv = dinv_ref[:, 0:1]
    esum = acc_ref[0] + acc_ref[1] - g2_ref[...]
    out2 = jax.nn.relu(esum * dinv + b2_ref[...])
    onehot = (batch_ref[0] == lax.broadcasted_iota(jnp.int32, (G, BN), 0)
              ).astype(jnp.float32)
    part = jnp.dot(onehot, out2, preferred_element_type=jnp.float32)

    @pl.when(i == 0)
    def _():
        p2_acc[...] = jnp.zeros_like(p2_acc)

    p2_acc[...] += part

    @pl.when(i == NB - 1)
    def _():
        p = jnp.concatenate([p1_ref[...], p2_acc[...]], axis=1)
        h = jnp.dot(p, wl1_ref[...], preferred_element_type=jnp.float32)
        h = jax.nn.relu(h + bl1_ref[...])
        h = jnp.dot(h, wl2_ref[...], preferred_element_type=jnp.float32)
        h = h + bl2_ref[...]
        m = jnp.max(h, axis=1, keepdims=True)
        lse = jnp.log(jnp.sum(jnp.exp(h - m), axis=1, keepdims=True))
        h_ref[...] = h
        lsm_ref[...] = h - m - lse


def _tc_stage3(acc2, g2, dinv8, b2r, batch3, p1, Wl1, bl1r, Wl2, bl2r):
    return pl.pallas_call(
        _tc3_body,
        grid=(NB,),
        in_specs=[
            pl.BlockSpec((NC, BN, D), lambda i: (0, i, 0)),
            pl.BlockSpec((BN, D), lambda i: (i, 0)),
            pl.BlockSpec((BN, 8), lambda i: (i, 0)),
            pl.BlockSpec((1, D), lambda i: (0, 0)),
            pl.BlockSpec((1, 1, BN), lambda i: (i, 0, 0)),
            pl.BlockSpec((G, D), lambda i: (0, 0)),
            pl.BlockSpec((2 * D, 2 * D), lambda i: (0, 0)),
            pl.BlockSpec((1, 2 * D), lambda i: (0, 0)),
            pl.BlockSpec((2 * D, 10), lambda i: (0, 0)),
            pl.BlockSpec((1, 10), lambda i: (0, 0)),
        ],
        out_specs=[
            pl.BlockSpec((G, 10), lambda i: (0, 0)),
            pl.BlockSpec((G, 10), lambda i: (0, 0)),
        ],
        out_shape=[
            jax.ShapeDtypeStruct((G, 10), jnp.float32),
            jax.ShapeDtypeStruct((G, 10), jnp.float32),
        ],
        scratch_shapes=[pltpu.VMEM((G, D), jnp.float32)],
    )(acc2, g2, dinv8, b2r, batch3, p1, Wl1, bl1r, Wl2, bl2r)


def kernel(x, edge_index, batch, W1, b1, W2, b2, Wl1, bl1, Wl2, bl2):
    batchp = jnp.pad(batch, (0, NPAD - N), constant_values=G)
    batch3 = jnp.reshape(batchp, (NB, 1, BN))

    xp = jnp.pad(x, ((0, NPAD - N), (0, 0)))

    eif = jnp.reshape(edge_index, (2 * E,))
    parts = jnp.reshape(_sc_degree(eif), (NC * NS, NPAD))
    dinv8 = _tc_stage0(parts)
    g1 = _tc_stage1(xp, W1, dinv8)
    acc1 = _sc_scatter(g1, eif)
    g2, p1 = _tc_stage2(acc1, g1, dinv8, W2, jnp.reshape(b1, (1, D)), batch3)
    acc2 = _sc_scatter(g2, eif)
    h, lsm = _tc_stage3(acc2, g2, dinv8, jnp.reshape(b2, (1, D)), batch3, p1,
                        Wl1, jnp.reshape(bl1, (1, 2 * D)), Wl2,
                        jnp.reshape(bl2, (1, 10)))
    return (h, lsm)
